# baseline (speedup 1.0000x reference)
import functools

import jax
import jax.numpy as jnp
from jax import lax
from jax.experimental import pallas as pl
from jax.experimental.pallas import tpu as pltpu
from jax.experimental.pallas import tpu_sc as plsc

F = 26
N = 1000
R = 16
B = 16384
L = 16
NC = 2
NS = 16
W = NC * NS
PW = (B * F) // W


def _sc_body(xt, yt, scf, std, out, xv_ref, yv_ref, tbl_ref, std_ref, out_ref):
    wid = lax.axis_index("s") * NC + lax.axis_index("c")
    start = wid * PW
    end = start + PW

    pltpu.sync_copy(xt.at[pl.ds(start, PW)], xv_ref)
    pltpu.sync_copy(yt.at[pl.ds(start, PW)], yv_ref)

    f0 = start // B
    f1 = (end - 1) // B
    mid = jnp.minimum(end, (f0 + 1) * B)

    for seg in range(2):
        f = f0 if seg == 0 else f1
        lo = start if seg == 0 else mid
        hi = mid if seg == 0 else end

        pltpu.sync_copy(scf.at[f], tbl_ref)
        pltpu.sync_copy(std.at[f], std_ref)

        base = lo - start

        @plsc.parallel_loop(base, base + (hi - lo), L, unroll=4)
        def group_body(off):
            xv = xv_ref[pl.ds(off, L)]
            yv = yv_ref[pl.ds(off, L)]
            acc = jnp.zeros((L,), jnp.float32)
            for r in range(R):
                gx = plsc.load_gather(tbl_ref, [xv + r * N])
                gy = plsc.load_gather(tbl_ref, [yv + r * N])
                h = gx * gy
                acc = acc + h * h
            sv = plsc.load_gather(std_ref, [xv])
            acc = acc + jnp.where(xv == yv, sv * sv, 0.0)
            out_ref[pl.ds(off, L)] = acc

    pltpu.sync_copy(out_ref, out.at[pl.ds(start, PW)])


@functools.partial(jax.jit)
def _sc_call(xt, yt, scf, std):
    mesh = plsc.VectorSubcoreMesh(core_axis_name="c", subcore_axis_name="s")
    k = pl.kernel(
        _sc_body,
        mesh=mesh,
        compiler_params=pltpu.CompilerParams(
            needs_layout_passes=False,
            skip_device_barrier=True,
            disable_bounds_checks=True,
            disable_semaphore_checks=True,
        ),
        out_type=jax.ShapeDtypeStruct((B * F,), jnp.float32),
        scratch_types=[
            pltpu.VMEM((PW,), jnp.int32),
            pltpu.VMEM((PW,), jnp.int32),
            pltpu.VMEM((N * R,), jnp.float32),
            pltpu.VMEM((N,), jnp.float32),
            pltpu.VMEM((PW,), jnp.float32),
        ],
    )
    return k(xt, yt, scf, std)


def kernel(x, y, sqrt_covar_factor, std):
    xt = x.astype(jnp.int32).T.reshape(B * F)
    yt = y.astype(jnp.int32).T.reshape(B * F)
    scf_t = sqrt_covar_factor.transpose(0, 2, 1).reshape(F, R * N)
    out_flat = _sc_call(xt, yt, scf_t, std)
    return out_flat.reshape(F, B).T


# speedup vs baseline: 1.1752x; 1.1752x over previous
import functools

import jax
import jax.numpy as jnp
from jax import lax
from jax.experimental import pallas as pl
from jax.experimental.pallas import tpu as pltpu
from jax.experimental.pallas import tpu_sc as plsc

F = 26
N = 1000
R = 16
B = 16384
L = 16
NC = 2
NS = 16
W = NC * NS
PW = (B * F) // W


def _sc_body(xt, yt, scf, std, out, xv_ref, yv_ref, tbl_ref, std_ref, out_ref):
    wid = lax.axis_index("s") * NC + lax.axis_index("c")
    start = wid * PW
    end = start + PW

    pltpu.sync_copy(xt.at[pl.ds(start, PW)], xv_ref)
    pltpu.sync_copy(yt.at[pl.ds(start, PW)], yv_ref)

    f0 = start // B
    f1 = (end - 1) // B
    mid = jnp.minimum(end, (f0 + 1) * B)

    for seg in range(2):
        f = f0 if seg == 0 else f1
        lo = start if seg == 0 else mid
        hi = mid if seg == 0 else end

        pltpu.sync_copy(scf.at[f], tbl_ref)
        pltpu.sync_copy(std.at[f], std_ref)

        base = lo - start

        @plsc.parallel_loop(base, base + (hi - lo), L, unroll=8)
        def group_body(off):
            xv = xv_ref[pl.ds(off, L)]
            yv = yv_ref[pl.ds(off, L)]
            acc = jnp.zeros((L,), jnp.float32)
            for r in range(R):
                gx = plsc.load_gather(tbl_ref, [xv + r * N])
                gy = plsc.load_gather(tbl_ref, [yv + r * N])
                acc = acc + (gx * gx) * (gy * gy)
            sv = plsc.load_gather(std_ref, [xv])
            acc = acc + jnp.where(xv == yv, sv * sv, 0.0)
            out_ref[pl.ds(off, L)] = acc

    pltpu.sync_copy(out_ref, out.at[pl.ds(start, PW)])


@functools.partial(jax.jit)
def _sc_call(xt, yt, scf, std):
    mesh = plsc.VectorSubcoreMesh(core_axis_name="c", subcore_axis_name="s")
    k = pl.kernel(
        _sc_body,
        mesh=mesh,
        compiler_params=pltpu.CompilerParams(
            needs_layout_passes=False,
            skip_device_barrier=True,
            disable_bounds_checks=True,
            disable_semaphore_checks=True,
        ),
        out_type=jax.ShapeDtypeStruct((B * F,), jnp.float32),
        scratch_types=[
            pltpu.VMEM((PW,), jnp.int32),
            pltpu.VMEM((PW,), jnp.int32),
            pltpu.VMEM((N * R,), jnp.float32),
            pltpu.VMEM((N,), jnp.float32),
            pltpu.VMEM((PW,), jnp.float32),
        ],
    )
    return k(xt, yt, scf, std)


def kernel(x, y, sqrt_covar_factor, std):
    xt = x.astype(jnp.int32).T.reshape(B * F)
    yt = y.astype(jnp.int32).T.reshape(B * F)
    scf_t = sqrt_covar_factor.transpose(0, 2, 1).reshape(F, R * N)
    out_flat = _sc_call(xt, yt, scf_t, std)
    return out_flat.reshape(F, B).T


# speedup vs baseline: 1.1862x; 1.0094x over previous
import functools

import jax
import jax.numpy as jnp
from jax import lax
from jax.experimental import pallas as pl
from jax.experimental.pallas import tpu as pltpu
from jax.experimental.pallas import tpu_sc as plsc

F = 26
N = 1000
R = 16
B = 16384
L = 16
NC = 2
NS = 16
W = NC * NS
PW = (B * F) // W


def _sc_body(xt, yt, scf, std, out, xv_ref, yv_ref, tbl_ref, std_ref, out_ref):
    wid = lax.axis_index("s") * NC + lax.axis_index("c")
    start = wid * PW
    end = start + PW

    pltpu.sync_copy(xt.at[pl.ds(start, PW)], xv_ref)
    pltpu.sync_copy(yt.at[pl.ds(start, PW)], yv_ref)

    f0 = start // B
    f1 = (end - 1) // B
    mid = jnp.minimum(end, (f0 + 1) * B)

    for seg in range(2):
        f = f0 if seg == 0 else f1
        lo = start if seg == 0 else mid
        hi = mid if seg == 0 else end

        pltpu.sync_copy(scf.at[f], tbl_ref)
        pltpu.sync_copy(std.at[f], std_ref)

        base = lo - start

        @plsc.parallel_loop(base, base + (hi - lo), L, unroll=4)
        def group_body(off):
            xv = xv_ref[pl.ds(off, L)]
            yv = yv_ref[pl.ds(off, L)]
            acc = jnp.zeros((L,), jnp.float32)
            for r in range(R):
                gx = plsc.load_gather(tbl_ref, [xv + r * N])
                gy = plsc.load_gather(tbl_ref, [yv + r * N])
                acc = acc + (gx * gx) * (gy * gy)
            sv = plsc.load_gather(std_ref, [xv])
            acc = acc + jnp.where(xv == yv, sv * sv, 0.0)
            out_ref[pl.ds(off, L)] = acc

    pltpu.sync_copy(out_ref, out.at[pl.ds(start, PW)])


@functools.partial(jax.jit)
def _sc_call(xt, yt, scf, std):
    mesh = plsc.VectorSubcoreMesh(core_axis_name="c", subcore_axis_name="s")
    k = pl.kernel(
        _sc_body,
        mesh=mesh,
        compiler_params=pltpu.CompilerParams(
            needs_layout_passes=False,
            skip_device_barrier=True,
            disable_bounds_checks=True,
            disable_semaphore_checks=True,
        ),
        out_type=jax.ShapeDtypeStruct((B * F,), jnp.float32),
        scratch_types=[
            pltpu.VMEM((PW,), jnp.int32),
            pltpu.VMEM((PW,), jnp.int32),
            pltpu.VMEM((N * R,), jnp.float32),
            pltpu.VMEM((N,), jnp.float32),
            pltpu.VMEM((PW,), jnp.float32),
        ],
    )
    return k(xt, yt, scf, std)


def kernel(x, y, sqrt_covar_factor, std):
    xt = x.astype(jnp.int32).T.reshape(B * F)
    yt = y.astype(jnp.int32).T.reshape(B * F)
    scf_t = sqrt_covar_factor.transpose(0, 2, 1).reshape(F, R * N)
    out_flat = _sc_call(xt, yt, scf_t, std)
    return out_flat.reshape(F, B).T


# speedup vs baseline: 1.2505x; 1.0542x over previous
import functools

import jax
import jax.numpy as jnp
from jax import lax
from jax.experimental import pallas as pl
from jax.experimental.pallas import tpu as pltpu
from jax.experimental.pallas import tpu_sc as plsc

F = 26
N = 1000
R = 16
B = 16384
L = 16
NC = 2
NS = 16
W = NC * NS
PW = (B * F) // W


def _sc_body(xt, yt, scf, std, out, xv_ref, yv_ref, tbl0_ref, tbl1_ref,
             std0_ref, std1_ref, out_ref, sem0, sem1):
    wid = lax.axis_index("s") * NC + lax.axis_index("c")
    start = wid * PW
    end = start + PW

    f0 = start // B
    f1 = (end - 1) // B
    mid = jnp.minimum(end, (f0 + 1) * B)

    c_x = pltpu.async_copy(xt.at[pl.ds(start, PW)], xv_ref, sem0)
    c_y = pltpu.async_copy(yt.at[pl.ds(start, PW)], yv_ref, sem0)
    c_t0 = pltpu.async_copy(scf.at[f0], tbl0_ref, sem0)
    c_s0 = pltpu.async_copy(std.at[f0], std0_ref, sem0)
    c_t1 = pltpu.async_copy(scf.at[f1], tbl1_ref, sem1)
    c_s1 = pltpu.async_copy(std.at[f1], std1_ref, sem1)

    def run_segment(tbl_ref, std_ref, base, length):
        @plsc.parallel_loop(base, base + length, L, unroll=4)
        def group_body(off):
            xv = xv_ref[pl.ds(off, L)]
            yv = yv_ref[pl.ds(off, L)]
            acc = jnp.zeros((L,), jnp.float32)
            for r in range(R):
                gx = plsc.load_gather(tbl_ref, [xv + r * N])
                gy = plsc.load_gather(tbl_ref, [yv + r * N])
                acc = acc + (gx * gx) * (gy * gy)
            sv = plsc.load_gather(std_ref, [xv])
            acc = acc + jnp.where(xv == yv, sv * sv, 0.0)
            out_ref[pl.ds(off, L)] = acc

    c_x.wait()
    c_y.wait()
    c_t0.wait()
    c_s0.wait()
    run_segment(tbl0_ref, std0_ref, 0, mid - start)
    c_t1.wait()
    c_s1.wait()
    run_segment(tbl1_ref, std1_ref, mid - start, end - mid)

    pltpu.sync_copy(out_ref, out.at[pl.ds(start, PW)])


@functools.partial(jax.jit)
def _sc_call(xt, yt, scf, std):
    mesh = plsc.VectorSubcoreMesh(core_axis_name="c", subcore_axis_name="s")
    k = pl.kernel(
        _sc_body,
        mesh=mesh,
        compiler_params=pltpu.CompilerParams(
            needs_layout_passes=False,
            skip_device_barrier=True,
            disable_bounds_checks=True,
            disable_semaphore_checks=True,
        ),
        out_type=jax.ShapeDtypeStruct((B * F,), jnp.float32),
        scratch_types=[
            pltpu.VMEM((PW,), jnp.int32),
            pltpu.VMEM((PW,), jnp.int32),
            pltpu.VMEM((N * R,), jnp.float32),
            pltpu.VMEM((N * R,), jnp.float32),
            pltpu.VMEM((N,), jnp.float32),
            pltpu.VMEM((N,), jnp.float32),
            pltpu.VMEM((PW,), jnp.float32),
            pltpu.SemaphoreType.DMA,
            pltpu.SemaphoreType.DMA,
        ],
    )
    return k(xt, yt, scf, std)


def kernel(x, y, sqrt_covar_factor, std):
    xt = x.astype(jnp.int32).T.reshape(B * F)
    yt = y.astype(jnp.int32).T.reshape(B * F)
    scf_t = sqrt_covar_factor.transpose(0, 2, 1).reshape(F, R * N)
    out_flat = _sc_call(xt, yt, scf_t, std)
    return out_flat.reshape(F, B).T
